# baseline (speedup 1.0000x reference)
import jax
import jax.numpy as jnp
from jax import lax
from jax.experimental import pallas as pl
from jax.experimental.pallas import tpu as pltpu
from jax.experimental.pallas import tpu_sc as plsc

D = 1024
V = 100000
VH = V // 2
CGW = 12512
CG_OWN = 12496
NRG = 4
RPT = D // NRG
NBUF = 8
NG = RPT
NS = RPT // 16
AW = 3136
NJ = CGW // 16
NJA = AW // 16


def _sc_body(x_hbm, w_hbm, b_hbm, vals_hbm, idx_hbm,
             xv, acc, *rest):
    wbufs = rest[:NBUF]
    rd, st_v, st_i, shacc_all = rest[NBUF:NBUF + 4]
    sems = rest[NBUF + 4:]
    cid = lax.axis_index("c")
    sid = lax.axis_index("s")
    wid = cid * 16 + sid
    rgrp = lax.rem(sid, NRG)
    cgrp = sid // NRG
    rbase = rgrp * RPT
    half = cid * VH
    cg_lo = half + cgrp * CG_OWN

    pltpu.sync_copy(x_hbm, xv)

    @pl.when(rgrp == 0)
    def _():
        pltpu.sync_copy(b_hbm.at[pl.ds(cg_lo, CGW)], acc)

    @pl.when(rgrp != 0)
    def _():
        @plsc.parallel_loop(0, NJ, unroll=4)
        def _zero(j):
            acc[pl.ds(j * 16, 16)] = jnp.zeros((16,), jnp.float32)

    def start(g, bi):
        pltpu.async_copy(
            w_hbm.at[rbase + g, pl.ds(cg_lo, CGW)], wbufs[bi], sems[bi])

    def wait_g(g, bi):
        pltpu.make_async_copy(
            w_hbm.at[rbase + g, pl.ds(cg_lo, CGW)],
            wbufs[bi], sems[bi]).wait()

    for bi in range(NBUF):
        start(bi, bi)

    def sbody(s, _):
        xg = xv[pl.ds(rbase + s * 16, 16)]
        xs = [xg[i] for i in range(16)]
        for k in range(16):
            g = s * 16 + k
            bi = k % NBUF
            wait_g(g, bi)
            xk = xs[k]
            buf = wbufs[bi]

            @plsc.parallel_loop(0, NJ, unroll=4)
            def _fma(j):
                sl = pl.ds(j * 16, 16)
                plsc.addupdate(acc.at[sl], xk * buf[sl])

            @pl.when(g + NBUF < NG)
            def _():
                start(g + NBUF, bi)
        return 0

    lax.fori_loop(0, NS, sbody, 0)

    pltpu.sync_copy(acc, shacc_all.at[sid])
    plsc.subcore_barrier()

    @pl.when(rgrp == 0)
    def _():
        for r in range(1, NRG):
            pltpu.sync_copy(shacc_all.at[cgrp * NRG + r], wbufs[r - 1])

        @plsc.parallel_loop(0, NJ, unroll=4)
        def _red(j):
            sl = pl.ds(j * 16, 16)
            plsc.addupdate(acc.at[sl],
                           (wbufs[0][sl] + wbufs[1][sl]) + wbufs[2][sl])

        pltpu.sync_copy(acc, shacc_all.at[sid])

    plsc.subcore_barrier()

    wblk = jnp.where(cgrp == NRG - 1, CG_OWN + 16, CG_OWN)
    alo = ((rgrp * (wblk - AW)) // (NRG - 1)) // 8 * 8
    pltpu.sync_copy(shacc_all.at[cgrp * NRG, pl.ds(alo, AW)], rd)
    gbase = cg_lo + alo

    def rbody(j, carry):
        vm, vi = carry
        v = rd[pl.ds(j * 16, 16)]
        col = gbase + j * 16 + lax.iota(jnp.int32, 16)
        upd = v > vm
        return jnp.where(upd, v, vm), jnp.where(upd, col, vi)

    vm0 = rd[pl.ds(0, 16)]
    vi0 = gbase + lax.iota(jnp.int32, 16)
    vm, vi = lax.fori_loop(1, NJA, rbody, (vm0, vi0))

    st_v[...] = vm
    st_i[...] = vi
    pltpu.sync_copy(st_v, vals_hbm.at[wid])
    pltpu.sync_copy(st_i, idx_hbm.at[wid])


_sc_top1 = pl.kernel(
    _sc_body,
    out_type=[
        jax.ShapeDtypeStruct((32, 16), jnp.float32),
        jax.ShapeDtypeStruct((32, 16), jnp.int32),
    ],
    mesh=plsc.VectorSubcoreMesh(core_axis_name="c", subcore_axis_name="s"),
    compiler_params=pltpu.CompilerParams(use_tc_tiling_on_sc=False),
    scratch_types=[
        pltpu.VMEM((D,), jnp.float32),
        pltpu.VMEM((CGW,), jnp.float32),
    ] + [pltpu.VMEM((CGW,), jnp.float32) for _ in range(NBUF)] + [
        pltpu.VMEM((AW,), jnp.float32),
        pltpu.VMEM((16,), jnp.float32),
        pltpu.VMEM((16,), jnp.int32),
        pltpu.VMEM_SHARED((16, CGW), jnp.float32),
    ] + [pltpu.SemaphoreType.DMA for _ in range(NBUF)],
)


def _merge_body(vals_ref, idx_ref, out_ref):
    m = jnp.max(vals_ref[...])
    out_ref[0] = jnp.min(jnp.where(vals_ref[...] == m, idx_ref[...], V))


def kernel(x, W, b):
    vals, idx = _sc_top1(x[0], W, b)
    topk_id = pl.pallas_call(
        _merge_body,
        out_specs=pl.BlockSpec(memory_space=pltpu.SMEM),
        out_shape=jax.ShapeDtypeStruct((1,), jnp.int32),
    )(vals, idx)
    return topk_id


# speedup vs baseline: 1.0448x; 1.0448x over previous
import jax
import jax.numpy as jnp
from jax import lax
from jax.experimental import pallas as pl
from jax.experimental.pallas import tpu as pltpu
from jax.experimental.pallas import tpu_sc as plsc

D = 1024
V = 100000

BV = 2048
NBT = 34

VS = 67968
VSC = V - VS
NW = 32
CW = 1024
R = 16
NG = D // R
NJ = CW // 16


def _tc_body(x_ref, w_ref, b_ref, vmax_out, vidx_out, vmax, vidx):
    j = pl.program_id(0)
    logits = jnp.dot(x_ref[...], w_ref[...],
                     preferred_element_type=jnp.float32) + b_ref[...]
    col = jax.lax.broadcasted_iota(jnp.int32, (1, BV), 1) + j * BV

    @pl.when(j == 0)
    def _():
        vmax[...] = logits
        vidx[...] = col

    @pl.when(j > 0)
    def _():
        upd = logits > vmax[...]
        vmax[...] = jnp.where(upd, logits, vmax[...])
        vidx[...] = jnp.where(upd, col, vidx[...])

    @pl.when(j == NBT - 1)
    def _():
        vmax_out[...] = vmax[...]
        vidx_out[...] = vidx[...]


def _sc_body(x_hbm, w_hbm, b_hbm, vals_hbm, idx_hbm,
             xv, acc, wb0, wb1, st_v, st_i, sem0, sem1):
    cid = lax.axis_index("c")
    sid = lax.axis_index("s")
    wid = sid * 2 + cid
    lo = VS + ((wid * (VSC - CW)) // (NW - 1)) // 8 * 8

    pltpu.sync_copy(x_hbm, xv)
    pltpu.sync_copy(b_hbm.at[pl.ds(lo, CW)], acc)

    def start(g, buf, sem):
        pltpu.async_copy(
            w_hbm.at[pl.ds(g * R, R), pl.ds(lo, CW)], buf, sem)

    def wait_g(g, buf, sem):
        pltpu.make_async_copy(
            w_hbm.at[pl.ds(g * R, R), pl.ds(lo, CW)], buf, sem).wait()

    start(0, wb0, sem0)
    start(1, wb1, sem1)

    def gbody(t, _):
        for bi, (buf, sem) in enumerate(((wb0, sem0), (wb1, sem1))):
            g = 2 * t + bi
            wait_g(g, buf, sem)
            xg = xv[pl.ds(g * R, 16)]
            xs = [xg[r] for r in range(R)]

            @plsc.parallel_loop(0, NJ, unroll=4)
            def _fma(j):
                sl = pl.ds(j * 16, 16)
                parts = []
                for c in range(4):
                    p = xs[4 * c] * buf[4 * c, sl]
                    for r in range(4 * c + 1, 4 * c + 4):
                        p = p + xs[r] * buf[r, sl]
                    parts.append(p)
                plsc.addupdate(acc.at[sl], (parts[0] + parts[1]) +
                               (parts[2] + parts[3]))

            @pl.when(g + 2 < NG)
            def _():
                start(g + 2, buf, sem)
        return 0

    lax.fori_loop(0, NG // 2, gbody, 0)

    def rbody(j, carry):
        vm, vi = carry
        v = acc[pl.ds(j * 16, 16)]
        col = lo + j * 16 + lax.iota(jnp.int32, 16)
        upd = v > vm
        return jnp.where(upd, v, vm), jnp.where(upd, col, vi)

    vm0 = acc[pl.ds(0, 16)]
    vi0 = lo + lax.iota(jnp.int32, 16)
    vm, vi = lax.fori_loop(1, NJ, rbody, (vm0, vi0))

    st_v[...] = vm
    st_i[...] = vi
    pltpu.sync_copy(st_v, vals_hbm.at[wid])
    pltpu.sync_copy(st_i, idx_hbm.at[wid])


_sc_top1 = pl.kernel(
    _sc_body,
    out_type=[
        jax.ShapeDtypeStruct((NW, 16), jnp.float32),
        jax.ShapeDtypeStruct((NW, 16), jnp.int32),
    ],
    mesh=plsc.VectorSubcoreMesh(core_axis_name="c", subcore_axis_name="s"),
    compiler_params=pltpu.CompilerParams(use_tc_tiling_on_sc=False),
    scratch_types=[
        pltpu.VMEM((D,), jnp.float32),
        pltpu.VMEM((CW,), jnp.float32),
        pltpu.VMEM((R, CW), jnp.float32),
        pltpu.VMEM((R, CW), jnp.float32),
        pltpu.VMEM((16,), jnp.float32),
        pltpu.VMEM((16,), jnp.int32),
        pltpu.SemaphoreType.DMA,
        pltpu.SemaphoreType.DMA,
    ],
)


def _merge_body(sv_ref, si_ref, tv_ref, ti_ref, out_ref):
    m1 = jnp.max(sv_ref[...])
    m2 = jnp.max(tv_ref[...])
    m = jnp.maximum(m1, m2)
    i1 = jnp.min(jnp.where(sv_ref[...] == m, si_ref[...], V))
    i2 = jnp.min(jnp.where(tv_ref[...] == m, ti_ref[...], V))
    out_ref[0] = jnp.minimum(i1, i2)


def kernel(x, W, b):
    x0 = x[0:1, :]
    b2 = b.reshape(1, V)
    sc_vals, sc_idx = _sc_top1(x[0], W, b)
    tc_vals, tc_idx = pl.pallas_call(
        _tc_body,
        grid=(NBT,),
        in_specs=[
            pl.BlockSpec((1, D), lambda j: (0, 0)),
            pl.BlockSpec((D, BV), lambda j: (0, j)),
            pl.BlockSpec((1, BV), lambda j: (0, j)),
        ],
        out_specs=[
            pl.BlockSpec((1, BV), lambda j: (0, 0)),
            pl.BlockSpec((1, BV), lambda j: (0, 0)),
        ],
        out_shape=[
            jax.ShapeDtypeStruct((1, BV), jnp.float32),
            jax.ShapeDtypeStruct((1, BV), jnp.int32),
        ],
        scratch_shapes=[
            pltpu.VMEM((1, BV), jnp.float32),
            pltpu.VMEM((1, BV), jnp.int32),
        ],
    )(x0, W, b2)
    topk_id = pl.pallas_call(
        _merge_body,
        out_specs=pl.BlockSpec(memory_space=pltpu.SMEM),
        out_shape=jax.ShapeDtypeStruct((1,), jnp.int32),
    )(sc_vals, sc_idx, tc_vals, tc_idx)
    return topk_id


# speedup vs baseline: 2.3515x; 2.2507x over previous
import jax
import jax.numpy as jnp
from jax.experimental import pallas as pl
from jax.experimental.pallas import tpu as pltpu

D = 1024
V = 100000
BV = 2048
NB = (V + BV - 1) // BV


def _top1_body(x_ref, w_ref, b_ref, out_ref, vmax, vidx):
    i = pl.program_id(0)
    logits = jnp.dot(x_ref[...], w_ref[...],
                     preferred_element_type=jnp.float32) + b_ref[...]
    col = jax.lax.broadcasted_iota(jnp.int32, (1, BV), 1) + i * BV
    logits = jnp.where(col < V, logits, -jnp.inf)

    @pl.when(i == 0)
    def _():
        vmax[...] = logits
        vidx[...] = col

    @pl.when(i > 0)
    def _():
        upd = logits > vmax[...]
        vmax[...] = jnp.where(upd, logits, vmax[...])
        vidx[...] = jnp.where(upd, col, vidx[...])

    @pl.when(i == NB - 1)
    def _():
        m = jnp.max(vmax[...])
        out_ref[0] = jnp.min(jnp.where(vmax[...] == m, vidx[...], V))


def kernel(x, W, b):
    x0 = x[0:1, :]
    b2 = b.reshape(1, V)
    topk_id = pl.pallas_call(
        _top1_body,
        grid=(NB,),
        in_specs=[
            pl.BlockSpec((1, D), lambda i: (0, 0)),
            pl.BlockSpec((D, BV), lambda i: (0, i)),
            pl.BlockSpec((1, BV), lambda i: (0, i)),
        ],
        out_specs=pl.BlockSpec(memory_space=pltpu.SMEM),
        out_shape=jax.ShapeDtypeStruct((1,), jnp.int32),
        scratch_shapes=[
            pltpu.VMEM((1, BV), jnp.float32),
            pltpu.VMEM((1, BV), jnp.int32),
        ],
    )(x0, W, b2)
    return topk_id
